# speedup vs baseline: 1.7063x; 1.7063x over previous
import functools

import jax
import jax.numpy as jnp
from jax import lax
from jax.experimental import pallas as pl
from jax.experimental.pallas import tpu as pltpu
from jax.experimental.pallas import tpu_sc as plsc

N = 10000
E = 320000
HID = 128

NC = 2
NS = 16
L = 16
NW = NC * NS

_MESH = plsc.VectorSubcoreMesh(core_axis_name="c", subcore_axis_name="s")


def _sc_gather_rows(table, idx, n_rows, chunk=64):
    D = table.shape[1]
    per_w = n_rows // NW
    n_chunks = per_w // chunk

    @functools.partial(
        pl.kernel,
        mesh=_MESH,
        out_type=jax.ShapeDtypeStruct((n_rows, D), table.dtype),
        scratch_types=[
            pltpu.VMEM((chunk,), jnp.int32),
            pltpu.VMEM((chunk, D), table.dtype),
            pltpu.SemaphoreType.DMA,
        ],
    )
    def k(table_hbm, idx_hbm, out_hbm, idx_v, rows_v, sem):
        wid = lax.axis_index("s") * NC + lax.axis_index("c")
        base = wid * per_w

        @pl.loop(0, n_chunks)
        def _(j):
            off = base + j * chunk
            pltpu.sync_copy(idx_hbm.at[pl.ds(off, chunk)], idx_v)
            pltpu.async_copy(table_hbm.at[idx_v], rows_v, sem).wait()
            pltpu.sync_copy(rows_v, out_hbm.at[pl.ds(off, chunk)])

    return k(table, idx)


def _embed_rows(table, idx):
    n_pad = 10240
    idx_p = jnp.concatenate(
        [idx.astype(jnp.int32), jnp.zeros((n_pad - N,), jnp.int32)])
    rows = _sc_gather_rows(table, idx_p, n_pad, chunk=64)
    return rows[:N]


def kernel(x, edge_index, edge_attr, pos, pert, ctrl, gene_table, pert_table,
           W1, a1s, a1d, b1, Wl1, bl1, W2, a2s, a2d, b2, Wl2, bl2,
           Wm1, bm1, Wm2, bm2):
    src, dst = edge_index[0], edge_index[1]

    pe = _embed_rows(gene_table, pos)
    nrm = jnp.linalg.norm(pe, axis=-1, keepdims=True)
    pe = pe * jnp.minimum(1.0, 1.0 / jnp.maximum(nrm, 1e-7))
    pert_e = _embed_rows(pert_table, pert)

    def gat(h_in, W, a_s, a_d, b):
        h = h_in @ W
        loop = jnp.arange(N, dtype=src.dtype)
        s = jnp.concatenate([src, loop])
        d = jnp.concatenate([dst, loop])
        e = jax.nn.leaky_relu(jnp.take(h @ a_s, s) + jnp.take(h @ a_d, d), 0.2)
        ex = jnp.exp(e)
        den = jax.ops.segment_sum(ex, d, num_segments=N)
        num = jax.ops.segment_sum(ex[:, None] * jnp.take(h, s, axis=0), d,
                                  num_segments=N)
        return num / (den[:, None] + 1e-16) + b

    h0 = jnp.concatenate([x, pe], axis=1)
    h = gat(h0, W1, a1s, a1d, b1) + (h0 @ Wl1 + bl1)
    h = jax.nn.relu(h)
    h = gat(h, W2, a2s, a2d, b2) + (h @ Wl2 + bl2)
    f = jnp.concatenate([h, ctrl, pert_e], axis=1)
    o = jax.nn.relu(f @ Wm1 + bm1) @ Wm2 + bm2
    return jax.nn.relu(o)[:, 0]


# speedup vs baseline: 31.5628x; 18.4978x over previous
import dataclasses
import functools

import jax
import jax.numpy as jnp
from jax import lax
from jax.experimental import pallas as pl
from jax.experimental.pallas import tpu as pltpu
from jax.experimental.pallas import tpu_sc as plsc

N = 10000
E = 320000
HID = 128

NC = 2
NS = 16
L = 16
NW = NC * NS

_MESH = plsc.VectorSubcoreMesh(core_axis_name="c", subcore_axis_name="s")

_CP = pltpu.CompilerParams()
if "needs_layout_passes" in pltpu.CompilerParams.__dataclass_fields__:
    _CP = dataclasses.replace(_CP, needs_layout_passes=False)


def _sc_gather_rows(table, idx, n_rows, chunk=64):
    D = table.shape[1]
    per_w = n_rows // NW
    n_chunks = per_w // chunk

    @functools.partial(
        pl.kernel,
        mesh=_MESH,
        out_type=jax.ShapeDtypeStruct((n_rows, D), table.dtype),
        scratch_types=[
            pltpu.VMEM((chunk,), jnp.int32),
            pltpu.VMEM((chunk, D), table.dtype),
            pltpu.SemaphoreType.DMA,
        ],
    )
    def k(table_hbm, idx_hbm, out_hbm, idx_v, rows_v, sem):
        wid = lax.axis_index("s") * NC + lax.axis_index("c")
        base = wid * per_w

        @pl.loop(0, n_chunks)
        def _(j):
            off = base + j * chunk
            pltpu.sync_copy(idx_hbm.at[pl.ds(off, chunk)], idx_v)
            pltpu.async_copy(table_hbm.at[idx_v], rows_v, sem).wait()
            pltpu.sync_copy(rows_v, out_hbm.at[pl.ds(off, chunk)])

    return k(table, idx)


def _embed_rows(table, idx):
    n_pad = 10240
    idx_p = jnp.concatenate(
        [idx.astype(jnp.int32), jnp.zeros((n_pad - N,), jnp.int32)])
    rows = _sc_gather_rows(table, idx_p, n_pad, chunk=64)
    return rows[:N]


EP = 331776
E_CHUNK = 128
N_CHUNKS_W = EP // (NW * E_CHUNK)
PER_W = EP // NW
NPAD = 10240
NDEN = 10016


def _sc_gat_edges(h, hs, hd, src_flat, dst_chunks):
    @functools.partial(
        pl.kernel,
        mesh=_MESH,
        out_type=[
            jax.ShapeDtypeStruct((2 * NPAD, HID), jnp.float32),
            jax.ShapeDtypeStruct((NW * NDEN,), jnp.float32),
        ],
        compiler_params=_CP,
        scratch_types=[
            pltpu.VMEM((N,), jnp.float32),
            pltpu.VMEM((N,), jnp.float32),
            pltpu.VMEM((NDEN,), jnp.float32),
            pltpu.VMEM((E_CHUNK,), jnp.int32),
            pltpu.VMEM((1, E_CHUNK), jnp.int32),
            pltpu.VMEM((E_CHUNK,), jnp.float32),
            pltpu.VMEM((E_CHUNK, HID), jnp.float32),
            pltpu.VMEM_SHARED((NPAD, HID), jnp.float32),
            pltpu.SemaphoreType.DMA,
        ],
    )
    def k(h_hbm, hs_hbm, hd_hbm, src_hbm, dchunk_hbm, num_hbm, den_hbm,
          hs_v, hd_v, den_v, src_v, dst_v, ex_v, rows_v, acc, sem):
        cidx = lax.axis_index("c")
        sid = lax.axis_index("s")
        wid = sid * NC + cidx
        base = wid * PER_W

        pltpu.sync_copy(hs_hbm, hs_v)
        pltpu.sync_copy(hd_hbm, hd_v)

        @pl.loop(0, NDEN // L)
        def _(t):
            den_v[pl.ds(t * L, L)] = jnp.zeros((L,), jnp.float32)

        @pl.loop(0, E_CHUNK)
        def _(r):
            for kk in range(HID // L):
                rows_v[r, pl.ds(kk * L, L)] = jnp.zeros((L,), jnp.float32)

        rows_per_tile = NPAD // NS
        @pl.loop(0, rows_per_tile // E_CHUNK)
        def _(t):
            pltpu.sync_copy(
                rows_v, acc.at[pl.ds(sid * rows_per_tile + t * E_CHUNK,
                                     E_CHUNK)])
        plsc.subcore_barrier()

        lane = lax.iota(jnp.int32, 16)
        nxt = jnp.minimum(lane + 1, 15)
        dnums = lax.GatherDimensionNumbers(
            offset_dims=(), collapsed_slice_dims=(0,), start_index_map=(0,))

        def shift_left(v):
            return lax.gather(v, nxt[:, None], dnums, slice_sizes=(1,),
                              mode=lax.GatherScatterMode.PROMISE_IN_BOUNDS)

        @pl.loop(0, N_CHUNKS_W)
        def _(j):
            off = base + j * E_CHUNK
            gcid = wid * N_CHUNKS_W + j
            pltpu.sync_copy(src_hbm.at[pl.ds(off, E_CHUNK)], src_v)
            cp = pltpu.async_copy(h_hbm.at[src_v], rows_v, sem)
            pltpu.sync_copy(dchunk_hbm.at[pl.ds(gcid, 1)], dst_v)
            for u in range(E_CHUNK // L):
                s16 = src_v[pl.ds(u * L, L)]
                d16 = dst_v[0, pl.ds(u * L, L)]
                dg = jnp.minimum(d16, N - 1)
                e = plsc.load_gather(hs_v, [s16]) + plsc.load_gather(hd_v, [dg])
                e = jnp.maximum(e, 0.2 * e)
                ex = jnp.exp(e)
                ex_v[pl.ds(u * L, L)] = ex
                kk, vv = plsc.sort_key_val(d16, ex)
                c = plsc.cumsum(vv)
                knx = shift_left(kk)
                bend = kk != knx
                plsc.addupdate_scatter(den_v, [kk], c,
                                       mask=bend | (lane == 15))
                plsc.addupdate_scatter(den_v, [knx], -c,
                                       mask=bend & (lane != 15))
            cp.wait()

            @pl.loop(0, E_CHUNK // L)
            def _(g):
                exg = ex_v[pl.ds(g * L, L)]
                for i in range(L):
                    r = g * L + i
                    exr = exg[i]
                    for kk in range(HID // L):
                        rows_v[r, pl.ds(kk * L, L)] = (
                            rows_v[r, pl.ds(kk * L, L)] * exr)

            pltpu.sync_copy(rows_v, acc.at[dst_v.at[0]], add=True)

        pltpu.sync_copy(den_v, den_hbm.at[pl.ds(wid * NDEN, NDEN)])
        plsc.subcore_barrier()
        row0 = sid * rows_per_tile
        pltpu.sync_copy(acc.at[pl.ds(row0, rows_per_tile)],
                        num_hbm.at[pl.ds(cidx * NPAD + row0, rows_per_tile)])

    return k(h, hs, hd, src_flat, dst_chunks)


def kernel(x, edge_index, edge_attr, pos, pert, ctrl, gene_table, pert_table,
           W1, a1s, a1d, b1, Wl1, bl1, W2, a2s, a2d, b2, Wl2, bl2,
           Wm1, bm1, Wm2, bm2):
    src, dst = edge_index[0], edge_index[1]

    pe = _embed_rows(gene_table, pos)
    nrm = jnp.linalg.norm(pe, axis=-1, keepdims=True)
    pe = pe * jnp.minimum(1.0, 1.0 / jnp.maximum(nrm, 1e-7))
    pert_e = _embed_rows(pert_table, pert)

    loop = jnp.arange(N, dtype=jnp.int32)
    npad_e = EP - (E + N)
    s_p = jnp.concatenate([src.astype(jnp.int32), loop,
                           jnp.zeros((npad_e,), jnp.int32)])
    d_p = jnp.concatenate([dst.astype(jnp.int32), loop,
                           jnp.full((npad_e,), N, jnp.int32)])
    dst_chunks = d_p.reshape(EP // E_CHUNK, E_CHUNK)

    def gat(h_in, W, a_s, a_d, b):
        h = h_in @ W
        hs = h @ a_s
        hd = h @ a_d
        nacc, dacc = _sc_gat_edges(h, hs, hd, s_p, dst_chunks)
        num = nacc[:N] + nacc[NPAD:NPAD + N]
        den = dacc.reshape(NW, NDEN).sum(axis=0)[:N]
        return num / (den[:, None] + 1e-16) + b

    h0 = jnp.concatenate([x, pe], axis=1)
    h = gat(h0, W1, a1s, a1d, b1) + (h0 @ Wl1 + bl1)
    h = jax.nn.relu(h)
    h = gat(h, W2, a2s, a2d, b2) + (h @ Wl2 + bl2)
    f = jnp.concatenate([h, ctrl, pert_e], axis=1)
    o = jax.nn.relu(f @ Wm1 + bm1) @ Wm2 + bm2
    return jax.nn.relu(o)[:, 0]
